# baseline (speedup 1.0000x reference)
import functools

import jax
import jax.numpy as jnp
import numpy as np
from jax import lax
from jax.experimental import pallas as pl
from jax.experimental.pallas import tpu as pltpu
from jax.experimental.pallas import tpu_sc as plsc

N = 4096
E = 65536
C = 32
NT = 16
B = 32
H = 64
TBL = C + NT
NC = 2
NS = 16
NW = NC * NS
EPW = E // NW
CH = EPW // 128
NPT = N // NS

_f32 = jnp.float32


def _sc_gather(table, src_idx, dst_idx):
    mesh = plsc.VectorSubcoreMesh(core_axis_name="c", subcore_axis_name="s",
                                  num_cores=NC, num_subcores=NS)

    @functools.partial(
        pl.kernel, mesh=mesh,
        compiler_params=pltpu.CompilerParams(use_tc_tiling_on_sc=False),
        out_type=[jax.ShapeDtypeStruct((E, 128), _f32),
                  jax.ShapeDtypeStruct((E, 128), _f32)],
        scratch_types=[pltpu.VMEM((CH, 128), jnp.int32),
                       pltpu.VMEM((2, EPW // 2, TBL), _f32),
                       pltpu.SemaphoreType.DMA,
                       pltpu.SemaphoreType.DMA],
    )
    def k(table_h, src_h, dst_h, gs_h, gd_h, idx_v, rows_v, sem_g, sem_w):
        c = lax.axis_index("c")
        s = lax.axis_index("s")
        wid = s * NC + c
        base = wid * EPW
        half_ch = CH // 2
        wb_pending = []
        for idx_h, out_h in ((src_h, gs_h), (dst_h, gd_h)):
            pltpu.sync_copy(idx_h.at[wid], idx_v)
            for half in range(2):
                if len(wb_pending) >= 2:
                    wb_pending.pop(0).wait()
                descs = []
                for kk in range(half_ch):
                    d = pltpu.make_async_copy(
                        table_h.at[idx_v.at[half * half_ch + kk]],
                        rows_v.at[half, pl.ds(kk * 128, 128)], sem_g)
                    d.start()
                    descs.append(d)
                for d in descs:
                    d.wait()
                w = pltpu.make_async_copy(
                    rows_v.at[half],
                    out_h.at[pl.ds(base + half * (EPW // 2), EPW // 2),
                             pl.ds(0, TBL)], sem_w)
                w.start()
                wb_pending.append(w)
        for d in wb_pending:
            d.wait()

    return k(table, src_idx, dst_idx)


def _sc_scatter(fe2, dst_idx, zeros):
    mesh = plsc.VectorSubcoreMesh(core_axis_name="c", subcore_axis_name="s",
                                  num_cores=NC, num_subcores=NS)

    @functools.partial(
        pl.kernel, mesh=mesh,
        compiler_params=pltpu.CompilerParams(use_tc_tiling_on_sc=False),
        out_type=[jax.ShapeDtypeStruct((N, 128), _f32),
                  jax.ShapeDtypeStruct((N, 128), _f32)],
        scratch_types=[pltpu.VMEM((CH, 128), jnp.int32),
                       pltpu.VMEM((EPW, C), _f32),
                       pltpu.VMEM((NPT, C), _f32),
                       pltpu.VMEM_SHARED((N, C), _f32),
                       pltpu.SemaphoreType.DMA],
    )
    def k(fe2_h, dst_h, zeros_h, o0_h, o1_h, idx_v, rows_v, stage_v, acc_sh,
          sem):
        c = lax.axis_index("c")
        s = lax.axis_index("s")
        wid = s * NC + c
        pltpu.sync_copy(zeros_h.at[pl.ds(s * NPT, NPT)], stage_v)
        pltpu.sync_copy(stage_v, acc_sh.at[pl.ds(s * NPT, NPT)])
        plsc.subcore_barrier()
        pltpu.sync_copy(dst_h.at[wid], idx_v)
        pltpu.sync_copy(fe2_h.at[pl.ds(wid * EPW, EPW)], rows_v)
        for kk in range(CH):
            pltpu.sync_copy(rows_v.at[pl.ds(kk * 128, 128)],
                            acc_sh.at[idx_v.at[kk]], add=True)
        plsc.subcore_barrier()
        pltpu.sync_copy(acc_sh.at[pl.ds(s * NPT, NPT)], stage_v)

        @pl.when(c == 0)
        def _():
            pltpu.sync_copy(stage_v, o0_h.at[pl.ds(s * NPT, NPT), pl.ds(0, C)])

        @pl.when(c == 1)
        def _():
            pltpu.sync_copy(stage_v, o1_h.at[pl.ds(s * NPT, NPT), pl.ds(0, C)])

    return k(fe2, dst_idx, zeros)


def _halve(p, to):
    w = p.shape[-1]
    while w > to:
        p = p[:, : w // 2] + p[:, w // 2:]
        w //= 2
    return p


EB = 1024
NTB = 1024


def _tc_table_body(fnt_ref, net_ref, wl1n_ref, tbl_ref):
    fn = jnp.transpose(fnt_ref[...])
    ne = jnp.transpose(net_ref[...])
    fn_l = (jnp.dot(fn, wl1n_ref[...], preferred_element_type=_f32)
            * np.float32(1.0 / np.sqrt(C)))
    tbl_ref[...] = jnp.concatenate([fn_l, ne], axis=1)


def _tc_table(fn_t, ne_t, wl1n):
    nb_spec = lambda w: pl.BlockSpec((NTB, w), lambda b: (b, 0))
    t_spec = lambda w: pl.BlockSpec((w, NTB), lambda b: (0, b))
    w_spec = lambda shape: pl.BlockSpec(shape, lambda b: (0, 0))
    return pl.pallas_call(
        _tc_table_body,
        grid=(N // NTB,),
        in_specs=[t_spec(C), t_spec(NT), w_spec((C, C))],
        out_specs=nb_spec(TBL),
        out_shape=jax.ShapeDtypeStruct((N, TBL), _f32),
    )(fn_t, ne_t, wl1n)


def _tc_edge_body(gs_ref, gd_ref, fet_ref, let_ref,
                  wl1e_ref, wm1_ref, w2p_ref, wsce_ref, wl2e_ref,
                  r64_ref, r32_ref, fe2_ref, feout_t_ref):
    rc = np.float32(1.0 / np.sqrt(C))
    gs = gs_ref[...]
    gd = gd_ref[...]
    fet = fet_ref[...]
    le = jnp.transpose(let_ref[...])
    fe_l = lax.dot_general(fet, wl1e_ref[...], (((0,), (0,)), ((), ())),
                           preferred_element_type=_f32) * rc
    f_cat = jnp.concatenate([gs[:, :C], gd[:, :C], fe_l], axis=1)
    h = jax.nn.silu(jnp.dot(le, wm1_ref[...], preferred_element_type=_f32)
                    * np.float32(1.0 / np.sqrt(B)))
    m = jnp.dot(f_cat, w2p_ref[...], preferred_element_type=_f32)
    hb = jnp.dot(h, r64_ref[...], preferred_element_type=_f32)
    pre = _halve(m * hb, C) * np.float32(1.0 / np.sqrt(H * 3 * C))
    fe2 = jax.nn.silu(pre)
    escal = jnp.concatenate([gs[:, C:TBL], gd[:, C:TBL], le], axis=1)
    v = jnp.dot(escal, wsce_ref[...], preferred_element_type=_f32)
    feb = lax.dot_general(fet, r32_ref[...], (((0,), (0,)), ((), ())),
                          preferred_element_type=_f32)
    sc_e = _halve(v * feb, C) * np.float32(1.0 / np.sqrt(C * (2 * NT + B)))
    fe2_ref[...] = fe2
    feout = jnp.dot(fe2, wl2e_ref[...], preferred_element_type=_f32) * rc + sc_e
    feout_t_ref[...] = jnp.transpose(feout)


def _tc_edge(g_src, g_dst, fe_t, le_t,
             wl1e, wm1, w2p, wsce, wl2e, r64, r32):
    grid = (E // EB,)
    eb_spec = lambda w: pl.BlockSpec((EB, w), lambda b: (b, 0))
    t_spec = lambda w: pl.BlockSpec((w, EB), lambda b: (0, b))
    w_spec = lambda shape: pl.BlockSpec(shape, lambda b: (0, 0))
    return pl.pallas_call(
        _tc_edge_body,
        grid=grid,
        in_specs=[eb_spec(128), eb_spec(128), t_spec(C), t_spec(B),
                  w_spec((C, C)), w_spec((B, H)),
                  w_spec((3 * C, H * C)), w_spec((2 * NT + B, C * C)),
                  w_spec((C, C)), w_spec((2 * NT + B, H * C)),
                  w_spec((C, C * C))],
        out_specs=[eb_spec(C), t_spec(C)],
        out_shape=[jax.ShapeDtypeStruct((E, C), _f32),
                   jax.ShapeDtypeStruct((C, E), _f32)],
    )(g_src, g_dst, fe_t, le_t,
      wl1e, wm1, w2p, wsce, wl2e, r64, r32)


NB = 512


def _tc_node_body(p0_ref, p1_ref, fnt_ref, net_ref, wl2n_ref, wscn_ref,
                  r16_ref, out_t_ref):
    fn2 = (p0_ref[:, :C] + p1_ref[:, :C]) * np.float32(1.0 / 16.0)
    fn = jnp.transpose(fnt_ref[...])
    ne = jnp.transpose(net_ref[...])
    u2 = jnp.dot(fn, wscn_ref[...], preferred_element_type=_f32)
    nb = jnp.dot(ne, r16_ref[...], preferred_element_type=_f32)
    sc_n = _halve(u2 * nb, C) * np.float32(1.0 / np.sqrt(C * NT))
    out = (jnp.dot(fn2, wl2n_ref[...], preferred_element_type=_f32)
           * np.float32(1.0 / np.sqrt(C)) + sc_n)
    out_t_ref[...] = jnp.transpose(out)


def _tc_node(p0, p1, fn_t, ne_t, wl2n, wscn, r16):
    grid = (N // NB,)
    nb_spec = lambda w: pl.BlockSpec((NB, w), lambda b: (b, 0))
    t_spec = lambda w: pl.BlockSpec((w, NB), lambda b: (0, b))
    w_spec = lambda shape: pl.BlockSpec(shape, lambda b: (0, 0))
    return pl.pallas_call(
        _tc_node_body,
        grid=grid,
        in_specs=[nb_spec(128), nb_spec(128), t_spec(C), t_spec(NT),
                  w_spec((C, C)), w_spec((C, NT * C)), w_spec((NT, NT * C))],
        out_specs=t_spec(C),
        out_shape=jax.ShapeDtypeStruct((C, N), _f32),
    )(p0, p1, fn_t, ne_t, wl2n, wscn, r16)


def kernel(f_node, f_edge, sh, node_emb, length_emb, edge_index,
           W_sc_node, W_sc_edge, W_lin1_node, W_lin1_edge,
           W_mlp1, W_mlp2, W_lin2_node, W_lin2_edge):
    src_idx = edge_index[0].reshape(NW, CH, 128)
    dst_idx = edge_index[1].reshape(NW, CH, 128)
    w2p = W_mlp2.reshape(H, 3 * C, C).transpose(1, 0, 2).reshape(3 * C, H * C)
    wsce = W_sc_edge.transpose(1, 0, 2).reshape(2 * NT + B, C * C)
    wscn = W_sc_node.reshape(C, NT * C)
    r64 = jnp.kron(jnp.eye(2 * NT + B, dtype=_f32), jnp.ones((1, C), _f32))
    r32 = jnp.kron(jnp.eye(C, dtype=_f32), jnp.ones((1, C), _f32))
    r16 = jnp.kron(jnp.eye(NT, dtype=_f32), jnp.ones((1, C), _f32))
    zeros = jnp.zeros((N, C), _f32)

    fn_t = f_node.T
    ne_t = node_emb.T
    fe_t = f_edge.T
    le_t = length_emb.T

    table = _tc_table(fn_t, ne_t, W_lin1_node)
    g_src, g_dst = _sc_gather(table, src_idx, dst_idx)
    fe2, feout_t = _tc_edge(g_src, g_dst, fe_t, le_t,
                            W_lin1_edge, W_mlp1, w2p, wsce,
                            W_lin2_edge, r64, r32)
    p0, p1 = _sc_scatter(fe2, dst_idx, zeros)
    fnout_t = _tc_node(p0, p1, fn_t, ne_t, W_lin2_node, wscn, r16)
    return (fnout_t.T, feout_t.T)


# speedup vs baseline: 1.0530x; 1.0530x over previous
import functools

import jax
import jax.numpy as jnp
import numpy as np
from jax import lax
from jax.experimental import pallas as pl
from jax.experimental.pallas import tpu as pltpu
from jax.experimental.pallas import tpu_sc as plsc

N = 4096
E = 65536
C = 32
NT = 16
B = 32
H = 64
TBL = C + NT
NC = 2
NS = 16
NW = NC * NS
EPW = E // NW
CH = EPW // 128
NPT = N // NS

_f32 = jnp.float32


def _sc_gather(table, src_idx, dst_idx):
    mesh = plsc.VectorSubcoreMesh(core_axis_name="c", subcore_axis_name="s",
                                  num_cores=NC, num_subcores=NS)

    @functools.partial(
        pl.kernel, mesh=mesh,
        compiler_params=pltpu.CompilerParams(use_tc_tiling_on_sc=False),
        out_type=[jax.ShapeDtypeStruct((E, 128), _f32),
                  jax.ShapeDtypeStruct((E, 128), _f32)],
        scratch_types=[pltpu.VMEM((CH, 128), jnp.int32),
                       pltpu.VMEM((2, EPW // 2, TBL), _f32),
                       pltpu.SemaphoreType.DMA,
                       pltpu.SemaphoreType.DMA],
    )
    def k(table_h, src_h, dst_h, gs_h, gd_h, idx_v, rows_v, sem_g, sem_w):
        c = lax.axis_index("c")
        s = lax.axis_index("s")
        wid = s * NC + c
        base = wid * EPW
        half_ch = CH // 2
        wb_pending = []
        for idx_h, out_h in ((src_h, gs_h), (dst_h, gd_h)):
            pltpu.sync_copy(idx_h.at[wid], idx_v)
            for half in range(2):
                if len(wb_pending) >= 2:
                    wb_pending.pop(0).wait()
                descs = []
                for kk in range(half_ch):
                    d = pltpu.make_async_copy(
                        table_h.at[idx_v.at[half * half_ch + kk]],
                        rows_v.at[half, pl.ds(kk * 128, 128)], sem_g)
                    d.start()
                    descs.append(d)
                for d in descs:
                    d.wait()
                w = pltpu.make_async_copy(
                    rows_v.at[half],
                    out_h.at[pl.ds(base + half * (EPW // 2), EPW // 2),
                             pl.ds(0, TBL)], sem_w)
                w.start()
                wb_pending.append(w)
        for d in wb_pending:
            d.wait()

    return k(table, src_idx, dst_idx)


def _sc_scatter(fe2, dst_idx, zeros):
    mesh = plsc.VectorSubcoreMesh(core_axis_name="c", subcore_axis_name="s",
                                  num_cores=NC, num_subcores=NS)

    @functools.partial(
        pl.kernel, mesh=mesh,
        compiler_params=pltpu.CompilerParams(use_tc_tiling_on_sc=False),
        out_type=[jax.ShapeDtypeStruct((N, 128), _f32),
                  jax.ShapeDtypeStruct((N, 128), _f32)],
        scratch_types=[pltpu.VMEM((CH, 128), jnp.int32),
                       pltpu.VMEM((EPW, C), _f32),
                       pltpu.VMEM((NPT, C), _f32),
                       pltpu.VMEM_SHARED((N, C), _f32),
                       pltpu.SemaphoreType.DMA],
    )
    def k(fe2_h, dst_h, zeros_h, o0_h, o1_h, idx_v, rows_v, stage_v, acc_sh,
          sem):
        c = lax.axis_index("c")
        s = lax.axis_index("s")
        wid = s * NC + c
        pltpu.sync_copy(zeros_h.at[pl.ds(s * NPT, NPT)], stage_v)
        pltpu.sync_copy(stage_v, acc_sh.at[pl.ds(s * NPT, NPT)])
        plsc.subcore_barrier()
        pltpu.sync_copy(dst_h.at[wid], idx_v)
        pltpu.sync_copy(fe2_h.at[pl.ds(wid * EPW, EPW)], rows_v)
        for kk in range(CH):
            pltpu.sync_copy(rows_v.at[pl.ds(kk * 128, 128)],
                            acc_sh.at[idx_v.at[kk]], add=True)
        plsc.subcore_barrier()
        pltpu.sync_copy(acc_sh.at[pl.ds(s * NPT, NPT)], stage_v)

        @pl.when(c == 0)
        def _():
            pltpu.sync_copy(stage_v, o0_h.at[pl.ds(s * NPT, NPT), pl.ds(0, C)])

        @pl.when(c == 1)
        def _():
            pltpu.sync_copy(stage_v, o1_h.at[pl.ds(s * NPT, NPT), pl.ds(0, C)])

    return k(fe2, dst_idx, zeros)


def _halve(p, to):
    w = p.shape[-1]
    while w > to:
        p = p[:, : w // 2] + p[:, w // 2:]
        w //= 2
    return p


EB = 1024
NTB = 1024


def _tc_table_body(fnt_ref, net_ref, wl1n_ref, tbl_ref):
    fn = jnp.transpose(fnt_ref[...])
    ne = jnp.transpose(net_ref[...])
    fn_l = (jnp.dot(fn, wl1n_ref[...], preferred_element_type=_f32)
            * np.float32(1.0 / np.sqrt(C)))
    tbl_ref[...] = jnp.concatenate([fn_l, ne], axis=1)


def _tc_table(fn_t, ne_t, wl1n):
    nb_spec = lambda w: pl.BlockSpec((NTB, w), lambda b: (b, 0))
    t_spec = lambda w: pl.BlockSpec((w, NTB), lambda b: (0, b))
    w_spec = lambda shape: pl.BlockSpec(shape, lambda b: (0, 0))
    return pl.pallas_call(
        _tc_table_body,
        grid=(N // NTB,),
        in_specs=[t_spec(C), t_spec(NT), w_spec((C, C))],
        out_specs=nb_spec(TBL),
        out_shape=jax.ShapeDtypeStruct((N, TBL), _f32),
    )(fn_t, ne_t, wl1n)


def _tc_edge_body(gs_ref, gd_ref, fet_ref, let_ref,
                  wl1e_ref, wm1_ref, w2p_ref, wsce_ref, wl2e_ref,
                  r64_ref, r32_ref, fe2_ref, feout_t_ref):
    rc = np.float32(1.0 / np.sqrt(C))
    gs = gs_ref[...]
    gd = gd_ref[...]
    fet = fet_ref[...]
    le = jnp.transpose(let_ref[...])
    fe_l = lax.dot_general(fet, wl1e_ref[...], (((0,), (0,)), ((), ())),
                           preferred_element_type=_f32) * rc
    f_cat = jnp.concatenate([gs[:, :C], gd[:, :C], fe_l], axis=1)
    h = jax.nn.silu(jnp.dot(le, wm1_ref[...], preferred_element_type=_f32)
                    * np.float32(1.0 / np.sqrt(B)))
    m = jnp.dot(f_cat, w2p_ref[...], preferred_element_type=_f32)
    hb = jnp.dot(h, r64_ref[...], preferred_element_type=_f32)
    pre = _halve(m * hb, C) * np.float32(1.0 / np.sqrt(H * 3 * C))
    fe2 = jax.nn.silu(pre)
    escal = jnp.concatenate([gs[:, C:TBL], gd[:, C:TBL], le], axis=1)
    v = jnp.dot(escal, wsce_ref[...], preferred_element_type=_f32)
    feb = lax.dot_general(fet, r32_ref[...], (((0,), (0,)), ((), ())),
                          preferred_element_type=_f32)
    sc_e = _halve(v * feb, C) * np.float32(1.0 / np.sqrt(C * (2 * NT + B)))
    q = EB // 4
    fe2_ref[...] = jnp.concatenate(
        [fe2[0:q], fe2[q:2 * q], fe2[2 * q:3 * q], fe2[3 * q:4 * q]], axis=1)
    feout = jnp.dot(fe2, wl2e_ref[...], preferred_element_type=_f32) * rc + sc_e
    feout_t_ref[...] = jnp.transpose(feout)


def _tc_edge(g_src, g_dst, fe_t, le_t,
             wl1e, wm1, w2p, wsce, wl2e, r64, r32):
    grid = (E // EB,)
    eb_spec = lambda w: pl.BlockSpec((EB, w), lambda b: (b, 0))
    t_spec = lambda w: pl.BlockSpec((w, EB), lambda b: (0, b))
    w_spec = lambda shape: pl.BlockSpec(shape, lambda b: (0, 0))
    return pl.pallas_call(
        _tc_edge_body,
        grid=grid,
        in_specs=[eb_spec(128), eb_spec(128), t_spec(C), t_spec(B),
                  w_spec((C, C)), w_spec((B, H)),
                  w_spec((3 * C, H * C)), w_spec((2 * NT + B, C * C)),
                  w_spec((C, C)), w_spec((2 * NT + B, H * C)),
                  w_spec((C, C * C))],
        out_specs=[pl.BlockSpec((EB // 4, 4 * C), lambda b: (b, 0)), t_spec(C)],
        out_shape=[jax.ShapeDtypeStruct((E // 4, 4 * C), _f32),
                   jax.ShapeDtypeStruct((C, E), _f32)],
    )(g_src, g_dst, fe_t, le_t,
      wl1e, wm1, w2p, wsce, wl2e, r64, r32)


NB = 512


def _tc_node_body(p0_ref, p1_ref, fnt_ref, net_ref, wl2n_ref, wscn_ref,
                  r16_ref, out_t_ref):
    fn2 = (p0_ref[:, :C] + p1_ref[:, :C]) * np.float32(1.0 / 16.0)
    fn = jnp.transpose(fnt_ref[...])
    ne = jnp.transpose(net_ref[...])
    u2 = jnp.dot(fn, wscn_ref[...], preferred_element_type=_f32)
    nb = jnp.dot(ne, r16_ref[...], preferred_element_type=_f32)
    sc_n = _halve(u2 * nb, C) * np.float32(1.0 / np.sqrt(C * NT))
    out = (jnp.dot(fn2, wl2n_ref[...], preferred_element_type=_f32)
           * np.float32(1.0 / np.sqrt(C)) + sc_n)
    out_t_ref[...] = jnp.transpose(out)


def _tc_node(p0, p1, fn_t, ne_t, wl2n, wscn, r16):
    grid = (N // NB,)
    nb_spec = lambda w: pl.BlockSpec((NB, w), lambda b: (b, 0))
    t_spec = lambda w: pl.BlockSpec((w, NB), lambda b: (0, b))
    w_spec = lambda shape: pl.BlockSpec(shape, lambda b: (0, 0))
    return pl.pallas_call(
        _tc_node_body,
        grid=grid,
        in_specs=[nb_spec(128), nb_spec(128), t_spec(C), t_spec(NT),
                  w_spec((C, C)), w_spec((C, NT * C)), w_spec((NT, NT * C))],
        out_specs=t_spec(C),
        out_shape=jax.ShapeDtypeStruct((C, N), _f32),
    )(p0, p1, fn_t, ne_t, wl2n, wscn, r16)


def kernel(f_node, f_edge, sh, node_emb, length_emb, edge_index,
           W_sc_node, W_sc_edge, W_lin1_node, W_lin1_edge,
           W_mlp1, W_mlp2, W_lin2_node, W_lin2_edge):
    src_idx = edge_index[0].reshape(NW, CH, 128)
    dst_idx = edge_index[1].reshape(NW, CH, 128)
    w2p = W_mlp2.reshape(H, 3 * C, C).transpose(1, 0, 2).reshape(3 * C, H * C)
    wsce = W_sc_edge.transpose(1, 0, 2).reshape(2 * NT + B, C * C)
    wscn = W_sc_node.reshape(C, NT * C)
    r64 = jnp.kron(jnp.eye(2 * NT + B, dtype=_f32), jnp.ones((1, C), _f32))
    r32 = jnp.kron(jnp.eye(C, dtype=_f32), jnp.ones((1, C), _f32))
    r16 = jnp.kron(jnp.eye(NT, dtype=_f32), jnp.ones((1, C), _f32))
    zeros = jnp.zeros((N, C), _f32)

    fn_t = f_node.T
    ne_t = node_emb.T
    fe_t = f_edge.T
    le_t = length_emb.T

    dst_perm_idx = (edge_index[1].reshape(E // EB, 4, EB // 4)
                    .transpose(0, 2, 1).reshape(NW, CH, 128))

    table = _tc_table(fn_t, ne_t, W_lin1_node)
    g_src, g_dst = _sc_gather(table, src_idx, dst_idx)
    fe2_wide, feout_t = _tc_edge(g_src, g_dst, fe_t, le_t,
                                 W_lin1_edge, W_mlp1, w2p, wsce,
                                 W_lin2_edge, r64, r32)
    fe2 = fe2_wide.reshape(E, C)
    p0, p1 = _sc_scatter(fe2, dst_perm_idx, zeros)
    fnout_t = _tc_node(p0, p1, fn_t, ne_t, W_lin2_node, wscn, r16)
    return (fnout_t.T, feout_t.T)
